# baseline (speedup 1.0000x reference)
import functools

import jax
import jax.numpy as jnp
from jax import lax
from jax.experimental import pallas as pl
from jax.experimental.pallas import tpu as pltpu
from jax.experimental.pallas import tpu_sc as plsc

N = 10000
DEG = 32
D = 128
OUT = 128
BN = 400
BN2 = 1000

R = 4
NBLK = N // R
NW = 32
BASE_BLKS = NBLK // NW
EXTRA = NBLK - BASE_BLKS * NW


def _sc_agg_body(neigh_hbm, agg_hbm, buf0, buf1, out0, out1,
                 sin0, sin1, sout0, sout1):
    w = lax.axis_index("s") * 2 + lax.axis_index("c")
    nblk = BASE_BLKS + jnp.where(w < EXTRA, 1, 0)
    base = w * BASE_BLKS + jnp.minimum(w, EXTRA)

    def start_in(blk, buf, sem):
        pltpu.async_copy(neigh_hbm.at[pl.ds(blk * R, R)], buf, sem)

    def wait_in(buf, sem):
        pltpu.make_async_copy(neigh_hbm.at[pl.ds(0, R)], buf, sem).wait()

    def start_out(blk, buf, sem):
        pltpu.async_copy(buf, agg_hbm.at[pl.ds(blk * R, R)], sem)

    def wait_out(buf, sem):
        pltpu.make_async_copy(buf, agg_hbm.at[pl.ds(0, R)], sem).wait()

    start_in(base, buf0, sin0)

    @pl.when(nblk > 1)
    def _():
        start_in(base + 1, buf1, sin1)

    npair = (BASE_BLKS + 2) // 2

    def do_block(b, buf, outb, s_in, s_out):
        valid = b < nblk

        @pl.when(valid)
        def _():
            wait_in(buf, s_in)

            for r in range(R):
                accs = [buf[r, 0, pl.ds(16 * c, 16)] for c in range(8)]
                for k in range(1, DEG):
                    for c in range(8):
                        accs[c] = accs[c] + buf[r, k, pl.ds(16 * c, 16)]
                for c in range(8):
                    outb[r, pl.ds(16 * c, 16)] = accs[c]

            @pl.when(b >= 2)
            def _():
                wait_out(outb, s_out)

            start_out(base + b, outb, s_out)

            @pl.when(b + 2 < nblk)
            def _():
                start_in(base + b + 2, buf, s_in)

    def pair_body(p, _):
        do_block(2 * p, buf0, out0, sin0, sout0)
        do_block(2 * p + 1, buf1, out1, sin1, sout1)
        return 0

    lax.fori_loop(0, npair, pair_body, 0)

    @pl.when(nblk >= 1)
    def _():
        wait_out(out0, sout0)

    @pl.when(nblk >= 2)
    def _():
        wait_out(out1, sout1)


def _sc_aggregate(neigh_feats):
    mesh = plsc.VectorSubcoreMesh(core_axis_name="c", subcore_axis_name="s")
    f = functools.partial(
        pl.kernel,
        out_type=jax.ShapeDtypeStruct((N, D), jnp.float32),
        mesh=mesh,
        scratch_types=[
            pltpu.VMEM((R, DEG, D), jnp.float32),
            pltpu.VMEM((R, DEG, D), jnp.float32),
            pltpu.VMEM((R, D), jnp.float32),
            pltpu.VMEM((R, D), jnp.float32),
            pltpu.SemaphoreType.DMA,
            pltpu.SemaphoreType.DMA,
            pltpu.SemaphoreType.DMA,
            pltpu.SemaphoreType.DMA,
        ],
    )(_sc_agg_body)
    return f(neigh_feats)


def _pass1_body(agg_ref, self_ref, nn_ref, w_self_ref, b_self_ref,
                w_neigh_ref, b_neigh_ref, h_ref, s1_ref, s2_ref):
    nn = nn_ref[...]
    nn = jnp.where(nn == 0.0, 1.0, nn)
    agg = agg_ref[...] / nn
    self_h = jnp.dot(self_ref[...], w_self_ref[...],
                     preferred_element_type=jnp.float32) + b_self_ref[...]
    agg_h = jnp.dot(agg, w_neigh_ref[...],
                    preferred_element_type=jnp.float32) + b_neigh_ref[...]
    h = jnp.concatenate([self_h, agg_h], axis=1)
    h_ref[...] = h.astype(jnp.bfloat16)
    ps1 = jnp.sum(h, axis=0, keepdims=True)
    ps2 = jnp.sum(h * h, axis=0, keepdims=True)

    @pl.when(pl.program_id(0) == 0)
    def _init():
        s1_ref[...] = ps1
        s2_ref[...] = ps2

    @pl.when(pl.program_id(0) != 0)
    def _acc():
        s1_ref[...] += ps1
        s2_ref[...] += ps2


def _pass2_body(h_ref, s1_ref, s2_ref, gamma_ref, beta_ref, out_ref):
    mean = s1_ref[...] / N
    var = s2_ref[...] / N - mean * mean
    scale = gamma_ref[...] * jax.lax.rsqrt(var + 1e-3)
    shift = beta_ref[...] - mean * scale
    h = h_ref[...].astype(jnp.float32)
    out_ref[...] = jnp.maximum(h * scale + shift, 0.0)


def kernel(self_feats, neigh_feats, self_nneigh, neigh_nneigh,
           W_self, b_self, W_neigh, b_neigh, gamma, beta):
    nn2 = self_nneigh.reshape(N, 1)
    b_self2 = b_self.reshape(1, OUT)
    b_neigh2 = b_neigh.reshape(1, OUT)
    gamma2 = gamma.reshape(1, 2 * OUT)
    beta2 = beta.reshape(1, 2 * OUT)

    agg_sum = _sc_aggregate(neigh_feats)

    grid = N // BN
    h, s1, s2 = pl.pallas_call(
        _pass1_body,
        grid=(grid,),
        in_specs=[
            pl.BlockSpec((BN, D), lambda i: (i, 0)),
            pl.BlockSpec((BN, D), lambda i: (i, 0)),
            pl.BlockSpec((BN, 1), lambda i: (i, 0)),
            pl.BlockSpec((D, OUT), lambda i: (0, 0)),
            pl.BlockSpec((1, OUT), lambda i: (0, 0)),
            pl.BlockSpec((D, OUT), lambda i: (0, 0)),
            pl.BlockSpec((1, OUT), lambda i: (0, 0)),
        ],
        out_specs=[
            pl.BlockSpec((BN, 2 * OUT), lambda i: (i, 0)),
            pl.BlockSpec((1, 2 * OUT), lambda i: (0, 0)),
            pl.BlockSpec((1, 2 * OUT), lambda i: (0, 0)),
        ],
        out_shape=[
            jax.ShapeDtypeStruct((N, 2 * OUT), jnp.bfloat16),
            jax.ShapeDtypeStruct((1, 2 * OUT), jnp.float32),
            jax.ShapeDtypeStruct((1, 2 * OUT), jnp.float32),
        ],
    )(agg_sum, self_feats, nn2, W_self, b_self2, W_neigh, b_neigh2)

    out = pl.pallas_call(
        _pass2_body,
        grid=(N // BN2,),
        in_specs=[
            pl.BlockSpec((BN2, 2 * OUT), lambda i: (i, 0)),
            pl.BlockSpec((1, 2 * OUT), lambda i: (0, 0)),
            pl.BlockSpec((1, 2 * OUT), lambda i: (0, 0)),
            pl.BlockSpec((1, 2 * OUT), lambda i: (0, 0)),
            pl.BlockSpec((1, 2 * OUT), lambda i: (0, 0)),
        ],
        out_specs=pl.BlockSpec((BN2, 2 * OUT), lambda i: (i, 0)),
        out_shape=jax.ShapeDtypeStruct((N, 2 * OUT), jnp.float32),
    )(h, s1, s2, gamma2, beta2)
    return out


# speedup vs baseline: 4.6058x; 4.6058x over previous
import jax
import jax.numpy as jnp
from jax.experimental import pallas as pl
from jax.experimental.pallas import tpu as pltpu

N = 10000
DEG = 32
D = 128
OUT = 128
BN = 400
P = N // BN


def _fused_body(neigh_ref, self_ref, nn_ref, w_self_ref, b_self_ref,
                w_neigh_ref, b_neigh_ref, gamma_ref, beta_ref,
                out_ref, s1_ref, s2_ref, h_scr):
    i = pl.program_id(0)

    @pl.when(i < P)
    def _compute():
        neigh = neigh_ref[...]
        agg = jnp.sum(neigh, axis=1)
        nn = nn_ref[...]
        nn = jnp.where(nn == 0.0, 1.0, nn)
        agg = agg / nn
        self_h = jnp.dot(self_ref[...], w_self_ref[...],
                         preferred_element_type=jnp.float32) + b_self_ref[...]
        agg_h = jnp.dot(agg, w_neigh_ref[...],
                        preferred_element_type=jnp.float32) + b_neigh_ref[...]
        h = jnp.concatenate([self_h, agg_h], axis=1)
        h_scr[pl.ds(i * BN, BN), :] = h.astype(jnp.bfloat16)
        ps1 = jnp.sum(h, axis=0, keepdims=True)
        ps2 = jnp.sum(h * h, axis=0, keepdims=True)

        @pl.when(i == 0)
        def _init():
            s1_ref[...] = ps1
            s2_ref[...] = ps2

        @pl.when(i != 0)
        def _acc():
            s1_ref[...] += ps1
            s2_ref[...] += ps2

    @pl.when(i >= P)
    def _normalize():
        j = i - P
        mean = s1_ref[...] / N
        var = s2_ref[...] / N - mean * mean
        scale = gamma_ref[...] * jax.lax.rsqrt(var + 1e-3)
        shift = beta_ref[...] - mean * scale
        h = h_scr[pl.ds(j * BN, BN), :].astype(jnp.float32)
        out_ref[...] = jnp.maximum(h * scale + shift, 0.0)


def kernel(self_feats, neigh_feats, self_nneigh, neigh_nneigh,
           W_self, b_self, W_neigh, b_neigh, gamma, beta):
    nn2 = self_nneigh.reshape(N, 1)
    b_self2 = b_self.reshape(1, OUT)
    b_neigh2 = b_neigh.reshape(1, OUT)
    gamma2 = gamma.reshape(1, 2 * OUT)
    beta2 = beta.reshape(1, 2 * OUT)

    out, s1, s2 = pl.pallas_call(
        _fused_body,
        grid=(2 * P,),
        in_specs=[
            pl.BlockSpec((BN, DEG, D), lambda i: (jnp.minimum(i, P - 1), 0, 0)),
            pl.BlockSpec((BN, D), lambda i: (jnp.minimum(i, P - 1), 0)),
            pl.BlockSpec((BN, 1), lambda i: (jnp.minimum(i, P - 1), 0)),
            pl.BlockSpec((D, OUT), lambda i: (0, 0)),
            pl.BlockSpec((1, OUT), lambda i: (0, 0)),
            pl.BlockSpec((D, OUT), lambda i: (0, 0)),
            pl.BlockSpec((1, OUT), lambda i: (0, 0)),
            pl.BlockSpec((1, 2 * OUT), lambda i: (0, 0)),
            pl.BlockSpec((1, 2 * OUT), lambda i: (0, 0)),
        ],
        out_specs=[
            pl.BlockSpec((BN, 2 * OUT), lambda i: (jnp.maximum(i - P, 0), 0)),
            pl.BlockSpec((1, 2 * OUT), lambda i: (0, 0)),
            pl.BlockSpec((1, 2 * OUT), lambda i: (0, 0)),
        ],
        out_shape=[
            jax.ShapeDtypeStruct((N, 2 * OUT), jnp.float32),
            jax.ShapeDtypeStruct((1, 2 * OUT), jnp.float32),
            jax.ShapeDtypeStruct((1, 2 * OUT), jnp.float32),
        ],
        scratch_shapes=[pltpu.VMEM((N, 2 * OUT), jnp.bfloat16)],
    )(neigh_feats, self_feats, nn2, W_self, b_self2, W_neigh, b_neigh2,
      gamma2, beta2)
    return out


# speedup vs baseline: 4.9629x; 1.0775x over previous
import jax
import jax.numpy as jnp
from jax.experimental import pallas as pl
from jax.experimental.pallas import tpu as pltpu

N = 10000
DEG = 32
D = 128
OUT = 128
BN = 400
BN2 = 1000
P0 = N // BN
P1 = N // BN2


def _fused_body(neigh_ref, self_ref, nn_ref, w_self_ref, b_self_ref,
                w_neigh_ref, b_neigh_ref, gamma_ref, beta_ref,
                out_ref, s1_ref, s2_ref, h_scr):
    i = pl.program_id(0)

    @pl.when(i < P0)
    def _compute():
        neigh = neigh_ref[...]
        agg = jnp.sum(neigh, axis=1)
        nn = nn_ref[...]
        nn = jnp.where(nn == 0.0, 1.0, nn)
        agg = agg / nn
        self_h = jnp.dot(self_ref[...], w_self_ref[...],
                         preferred_element_type=jnp.float32) + b_self_ref[...]
        agg_h = jnp.dot(agg, w_neigh_ref[...],
                        preferred_element_type=jnp.float32) + b_neigh_ref[...]
        h = jnp.concatenate([self_h, agg_h], axis=1)
        h_scr[pl.ds(i * BN, BN), :] = h
        ps1 = jnp.sum(h, axis=0, keepdims=True)
        ps2 = jnp.sum(h * h, axis=0, keepdims=True)

        @pl.when(i == 0)
        def _init():
            s1_ref[...] = ps1
            s2_ref[...] = ps2

        @pl.when(i != 0)
        def _acc():
            s1_ref[...] += ps1
            s2_ref[...] += ps2

    @pl.when(i >= P0)
    def _normalize():
        j = i - P0
        mean = s1_ref[...] / N
        var = s2_ref[...] / N - mean * mean
        scale = gamma_ref[...] * jax.lax.rsqrt(var + 1e-3)
        shift = beta_ref[...] - mean * scale
        h = h_scr[pl.ds(j * BN2, BN2), :]
        out_ref[...] = jnp.maximum(h * scale + shift, 0.0)


def kernel(self_feats, neigh_feats, self_nneigh, neigh_nneigh,
           W_self, b_self, W_neigh, b_neigh, gamma, beta):
    nn2 = self_nneigh.reshape(N, 1)
    b_self2 = b_self.reshape(1, OUT)
    b_neigh2 = b_neigh.reshape(1, OUT)
    gamma2 = gamma.reshape(1, 2 * OUT)
    beta2 = beta.reshape(1, 2 * OUT)

    out, s1, s2 = pl.pallas_call(
        _fused_body,
        grid=(P0 + P1,),
        in_specs=[
            pl.BlockSpec((BN, DEG, D),
                         lambda i: (jnp.minimum(i, P0 - 1), 0, 0)),
            pl.BlockSpec((BN, D), lambda i: (jnp.minimum(i, P0 - 1), 0)),
            pl.BlockSpec((BN, 1), lambda i: (jnp.minimum(i, P0 - 1), 0)),
            pl.BlockSpec((D, OUT), lambda i: (0, 0)),
            pl.BlockSpec((1, OUT), lambda i: (0, 0)),
            pl.BlockSpec((D, OUT), lambda i: (0, 0)),
            pl.BlockSpec((1, OUT), lambda i: (0, 0)),
            pl.BlockSpec((1, 2 * OUT), lambda i: (0, 0)),
            pl.BlockSpec((1, 2 * OUT), lambda i: (0, 0)),
        ],
        out_specs=[
            pl.BlockSpec((BN2, 2 * OUT), lambda i: (jnp.maximum(i - P0, 0), 0)),
            pl.BlockSpec((1, 2 * OUT), lambda i: (0, 0)),
            pl.BlockSpec((1, 2 * OUT), lambda i: (0, 0)),
        ],
        out_shape=[
            jax.ShapeDtypeStruct((N, 2 * OUT), jnp.float32),
            jax.ShapeDtypeStruct((1, 2 * OUT), jnp.float32),
            jax.ShapeDtypeStruct((1, 2 * OUT), jnp.float32),
        ],
        scratch_shapes=[pltpu.VMEM((N, 2 * OUT), jnp.float32)],
    )(neigh_feats, self_feats, nn2, W_self, b_self2, W_neigh, b_neigh2,
      gamma2, beta2)
    return out
